# baseline (speedup 1.0000x reference)
import functools

import jax
import jax.numpy as jnp
from jax import lax
from jax.experimental import pallas as pl
from jax.experimental.pallas import tpu as pltpu
from jax.experimental.pallas import tpu_sc as plsc

N_CHANNELS = 4096
N_ENERGIES = 8192

NC = 2
NS = 16
NW = NC * NS
L = 16

SC_ROWS = N_CHANNELS
RPW = SC_ROWS // NW
R = 4
NG = RPW // R
NCHUNK = N_ENERGIES // L


def _flux_tc_kernel(params_ref, energies_ref, flux_ref):
    alpha = params_ref[0, 0]
    norm = params_ref[0, 1]
    oma = 1.0 - alpha
    e_lo = energies_ref[0, :]
    e_hi = energies_ref[1, :]
    flux_ref[0, :] = (norm / oma) * (
        jnp.exp(oma * jnp.log(e_hi)) - jnp.exp(oma * jnp.log(e_lo))
    )


def _compute_flux(params2d, energies):
    return pl.pallas_call(
        _flux_tc_kernel,
        in_specs=[
            pl.BlockSpec(memory_space=pltpu.SMEM),
            pl.BlockSpec((2, N_ENERGIES), lambda: (0, 0)),
        ],
        out_specs=pl.BlockSpec((1, N_ENERGIES), lambda: (0, 0)),
        out_shape=jax.ShapeDtypeStruct((1, N_ENERGIES), jnp.float32),
    )(params2d, energies)


def _sc_body(flux_hbm, tm_hbm, out_hbm, flux_v, buf_v, tbuf_v, out_v, sem0, sem1):
    wid = lax.axis_index("s") * NC + lax.axis_index("c")
    row0 = wid * RPW
    sems = (sem0, sem1)

    pltpu.sync_copy(flux_hbm, flux_v)
    pltpu.async_copy(tm_hbm.at[pl.ds(row0, R), :], buf_v.at[0], sem0)
    pltpu.async_copy(tm_hbm.at[pl.ds(row0 + R, R), :], buf_v.at[1], sem1)

    lane = lax.iota(jnp.int32, L)
    zero = jnp.zeros((L,), jnp.float32)

    def tile_loop(t, _):
        for sub in range(L // R):
            b = sub % 2
            g = t * (L // R) + sub
            pltpu.make_async_copy(
                tm_hbm.at[pl.ds(row0, R), :], buf_v.at[b], sems[b]
            ).wait()

            def chunk(i, accs, b=b):
                off = i * L
                fl = flux_v[pl.ds(off, L)]
                return tuple(
                    accs[r] + buf_v[b, r, pl.ds(off, L)] * fl for r in range(R)
                )

            accs = lax.fori_loop(0, NCHUNK, chunk, (zero,) * R, unroll=4)
            for r in range(R):
                tbuf_v[sub * R + r, :] = accs[r]

            @pl.when(g + 2 < NG)
            def _next(b=b, g=g):
                pltpu.async_copy(
                    tm_hbm.at[pl.ds(row0 + (g + 2) * R, R), :],
                    buf_v.at[b],
                    sems[b],
                )

        tot = zero
        for k in range(L):
            tot = tot + plsc.load_gather(tbuf_v, [lane, lane * 0 + k])
        out_v[pl.ds(t * L, L)] = jnp.maximum(tot, 1e-6)
        return 0

    lax.fori_loop(0, RPW // L, tile_loop, 0)

    pltpu.sync_copy(out_v, out_hbm.at[pl.ds(row0, RPW)])


_sc_matvec = functools.partial(
    pl.kernel,
    out_type=jax.ShapeDtypeStruct((SC_ROWS,), jnp.float32),
    mesh=plsc.VectorSubcoreMesh(core_axis_name="c", subcore_axis_name="s"),
    scratch_types=[
        pltpu.VMEM((N_ENERGIES,), jnp.float32),
        pltpu.VMEM((2, R, N_ENERGIES), jnp.float32),
        pltpu.VMEM((L, L), jnp.float32),
        pltpu.VMEM((RPW,), jnp.float32),
        pltpu.SemaphoreType.DMA,
        pltpu.SemaphoreType.DMA,
    ],
    compiler_params=pltpu.CompilerParams(needs_layout_passes=False),
)(_sc_body)


def kernel(parameters, energies, transfer_matrix):
    params2d = parameters.reshape(1, 2)
    flux = _compute_flux(params2d, energies).reshape(N_ENERGIES)
    out = _sc_matvec(flux, transfer_matrix)
    return out


# speedup vs baseline: 1.2615x; 1.2615x over previous
import functools

import jax
import jax.numpy as jnp
from jax import lax
from jax.experimental import pallas as pl
from jax.experimental.pallas import tpu as pltpu
from jax.experimental.pallas import tpu_sc as plsc

N_CHANNELS = 4096
N_ENERGIES = 8192

NC = 2
NS = 16
NW = NC * NS
L = 16

SC_ROWS = 1536
TC_ROWS = N_CHANNELS - SC_ROWS
BC = 256
RPW = SC_ROWS // NW
R = 4
NG = RPW // R
NCHUNK = N_ENERGIES // L


def _flux_tc_kernel(params_ref, energies_ref, flux_ref):
    alpha = params_ref[0, 0]
    norm = params_ref[0, 1]
    oma = 1.0 - alpha
    e_lo = energies_ref[0, :]
    e_hi = energies_ref[1, :]
    flux_ref[0, :] = (norm / oma) * (
        jnp.exp(oma * jnp.log(e_hi)) - jnp.exp(oma * jnp.log(e_lo))
    )


def _compute_flux(params2d, energies):
    return pl.pallas_call(
        _flux_tc_kernel,
        in_specs=[
            pl.BlockSpec(memory_space=pltpu.SMEM),
            pl.BlockSpec((2, N_ENERGIES), lambda: (0, 0)),
        ],
        out_specs=pl.BlockSpec((1, N_ENERGIES), lambda: (0, 0)),
        out_shape=jax.ShapeDtypeStruct((1, N_ENERGIES), jnp.float32),
    )(params2d, energies)


def _sc_body(flux_hbm, tm_hbm, out_hbm, flux_v, buf_v, tbuf_v, out_v, sem0, sem1):
    wid = lax.axis_index("s") * NC + lax.axis_index("c")
    row0 = wid * RPW
    sems = (sem0, sem1)

    pltpu.sync_copy(flux_hbm, flux_v)
    pltpu.async_copy(tm_hbm.at[pl.ds(row0, R), :], buf_v.at[0], sem0)
    pltpu.async_copy(tm_hbm.at[pl.ds(row0 + R, R), :], buf_v.at[1], sem1)

    lane = lax.iota(jnp.int32, L)
    zero = jnp.zeros((L,), jnp.float32)

    def tile_loop(t, _):
        for sub in range(L // R):
            b = sub % 2
            g = t * (L // R) + sub
            pltpu.make_async_copy(
                tm_hbm.at[pl.ds(row0, R), :], buf_v.at[b], sems[b]
            ).wait()

            def chunk(i, accs, b=b):
                off = i * L
                fl = flux_v[pl.ds(off, L)]
                return tuple(
                    accs[r] + buf_v[b, r, pl.ds(off, L)] * fl for r in range(R)
                )

            accs = lax.fori_loop(0, NCHUNK, chunk, (zero,) * R, unroll=4)
            for r in range(R):
                tbuf_v[sub * R + r, :] = accs[r]

            @pl.when(g + 2 < NG)
            def _next(b=b, g=g):
                pltpu.async_copy(
                    tm_hbm.at[pl.ds(row0 + (g + 2) * R, R), :],
                    buf_v.at[b],
                    sems[b],
                )

        tot = zero
        for k in range(L):
            tot = tot + plsc.load_gather(tbuf_v, [lane, lane * 0 + k])
        out_v[pl.ds(t * L, L)] = jnp.maximum(tot, 1e-6)
        return 0

    lax.fori_loop(0, RPW // L, tile_loop, 0)

    pltpu.sync_copy(out_v, out_hbm.at[pl.ds(row0, RPW)])


_sc_matvec = functools.partial(
    pl.kernel,
    out_type=jax.ShapeDtypeStruct((SC_ROWS,), jnp.float32),
    mesh=plsc.VectorSubcoreMesh(core_axis_name="c", subcore_axis_name="s"),
    scratch_types=[
        pltpu.VMEM((N_ENERGIES,), jnp.float32),
        pltpu.VMEM((2, R, N_ENERGIES), jnp.float32),
        pltpu.VMEM((L, L), jnp.float32),
        pltpu.VMEM((RPW,), jnp.float32),
        pltpu.SemaphoreType.DMA,
        pltpu.SemaphoreType.DMA,
    ],
    compiler_params=pltpu.CompilerParams(needs_layout_passes=False),
)(_sc_body)


def _tc_matvec_kernel(flux_ref, tm_ref, out_ref):
    flux = flux_ref[0, :].reshape(N_ENERGIES, 1)
    res = jnp.dot(tm_ref[...], flux, preferred_element_type=jnp.float32)
    out_ref[...] = jnp.maximum(res, 1e-6)


def _tc_matvec(flux2d, transfer_matrix):
    grid = TC_ROWS // BC
    off = SC_ROWS // BC
    return pl.pallas_call(
        _tc_matvec_kernel,
        grid=(grid,),
        in_specs=[
            pl.BlockSpec((1, N_ENERGIES), lambda i: (0, 0)),
            pl.BlockSpec((BC, N_ENERGIES), lambda i: (off + i, 0)),
        ],
        out_specs=pl.BlockSpec((BC, 1), lambda i: (i, 0)),
        out_shape=jax.ShapeDtypeStruct((TC_ROWS, 1), jnp.float32),
    )(flux2d, transfer_matrix)


def kernel(parameters, energies, transfer_matrix):
    params2d = parameters.reshape(1, 2)
    flux2d = _compute_flux(params2d, energies)
    sc_out = _sc_matvec(flux2d.reshape(N_ENERGIES), transfer_matrix)
    tc_out = _tc_matvec(flux2d, transfer_matrix).reshape(TC_ROWS)
    return jnp.concatenate([sc_out, tc_out])


# speedup vs baseline: 1.2623x; 1.0007x over previous
import functools

import jax
import jax.numpy as jnp
from jax import lax
from jax.experimental import pallas as pl
from jax.experimental.pallas import tpu as pltpu
from jax.experimental.pallas import tpu_sc as plsc

N_CHANNELS = 4096
N_ENERGIES = 8192

NC = 2
NS = 16
NW = NC * NS
L = 16

SC_ROWS = 1536
TC_ROWS = N_CHANNELS - SC_ROWS
BC = 256
RPW = SC_ROWS // NW
R = 4
NG = RPW // R
NCHUNK = N_ENERGIES // L


def _flux_tc_kernel(params_ref, energies_ref, flux_ref):
    alpha = params_ref[0, 0]
    norm = params_ref[0, 1]
    oma = 1.0 - alpha
    e_lo = energies_ref[0, :]
    e_hi = energies_ref[1, :]
    flux_ref[0, :] = (norm / oma) * (
        jnp.exp(oma * jnp.log(e_hi)) - jnp.exp(oma * jnp.log(e_lo))
    )


def _compute_flux(params2d, energies):
    return pl.pallas_call(
        _flux_tc_kernel,
        in_specs=[
            pl.BlockSpec(memory_space=pltpu.SMEM),
            pl.BlockSpec((2, N_ENERGIES), lambda: (0, 0)),
        ],
        out_specs=pl.BlockSpec((1, N_ENERGIES), lambda: (0, 0)),
        out_shape=jax.ShapeDtypeStruct((1, N_ENERGIES), jnp.float32),
    )(params2d, energies)


def _sc_body(flux_hbm, tm_hbm, out_hbm, flux_v, buf_v, tbuf_v, out_v, sem0, sem1):
    wid = lax.axis_index("s") * NC + lax.axis_index("c")
    row0 = wid * RPW
    sems = (sem0, sem1)

    pltpu.sync_copy(flux_hbm, flux_v)
    pltpu.async_copy(tm_hbm.at[pl.ds(row0, R), :], buf_v.at[0], sem0)
    pltpu.async_copy(tm_hbm.at[pl.ds(row0 + R, R), :], buf_v.at[1], sem1)

    lane = lax.iota(jnp.int32, L)
    zero = jnp.zeros((L,), jnp.float32)

    def tile_loop(t, _):
        for sub in range(L // R):
            b = sub % 2
            g = t * (L // R) + sub
            pltpu.make_async_copy(
                tm_hbm.at[pl.ds(row0, R), :], buf_v.at[b], sems[b]
            ).wait()

            def chunk(i, accs, b=b):
                off = i * L
                fl = flux_v[pl.ds(off, L)]
                return tuple(
                    accs[r] + buf_v[b, r, pl.ds(off, L)] * fl for r in range(R)
                )

            accs = lax.fori_loop(0, NCHUNK, chunk, (zero,) * R, unroll=4)
            for r in range(R):
                tbuf_v[sub * R + r, :] = accs[r]

            @pl.when(g + 2 < NG)
            def _next(b=b, g=g):
                pltpu.async_copy(
                    tm_hbm.at[pl.ds(row0 + (g + 2) * R, R), :],
                    buf_v.at[b],
                    sems[b],
                )

        tot = zero
        for k in range(L):
            tot = tot + plsc.load_gather(tbuf_v, [lane, lane * 0 + k])
        out_v[pl.ds(t * L, L)] = jnp.maximum(tot, 1e-6)
        return 0

    lax.fori_loop(0, RPW // L, tile_loop, 0)

    pltpu.sync_copy(out_v, out_hbm.at[pl.ds(row0, RPW)])


_sc_matvec = functools.partial(
    pl.kernel,
    out_type=jax.ShapeDtypeStruct((SC_ROWS,), jnp.float32),
    mesh=plsc.VectorSubcoreMesh(core_axis_name="c", subcore_axis_name="s"),
    scratch_types=[
        pltpu.VMEM((N_ENERGIES,), jnp.float32),
        pltpu.VMEM((2, R, N_ENERGIES), jnp.float32),
        pltpu.VMEM((L, L), jnp.float32),
        pltpu.VMEM((RPW,), jnp.float32),
        pltpu.SemaphoreType.DMA,
        pltpu.SemaphoreType.DMA,
    ],
    compiler_params=pltpu.CompilerParams(needs_layout_passes=False),
    cost_estimate=pl.CostEstimate(
        flops=2 * SC_ROWS * N_ENERGIES,
        transcendentals=0,
        bytes_accessed=SC_ROWS * N_ENERGIES * 4,
    ),
)(_sc_body)


def _tc_matvec_kernel(flux_ref, tm_ref, out_ref):
    flux = flux_ref[0, :].reshape(N_ENERGIES, 1)
    res = jnp.dot(tm_ref[...], flux, preferred_element_type=jnp.float32)
    out_ref[...] = jnp.maximum(res, 1e-6)


def _tc_matvec(flux2d, transfer_matrix):
    grid = TC_ROWS // BC
    off = SC_ROWS // BC
    return pl.pallas_call(
        _tc_matvec_kernel,
        grid=(grid,),
        in_specs=[
            pl.BlockSpec((1, N_ENERGIES), lambda i: (0, 0)),
            pl.BlockSpec((BC, N_ENERGIES), lambda i: (off + i, 0)),
        ],
        out_specs=pl.BlockSpec((BC, 1), lambda i: (i, 0)),
        out_shape=jax.ShapeDtypeStruct((TC_ROWS, 1), jnp.float32),
    )(flux2d, transfer_matrix)


def kernel(parameters, energies, transfer_matrix):
    params2d = parameters.reshape(1, 2)
    flux2d = _compute_flux(params2d, energies)
    sc_out = _sc_matvec(flux2d.reshape(N_ENERGIES), transfer_matrix)
    tc_out = _tc_matvec(flux2d, transfer_matrix).reshape(TC_ROWS)
    return jnp.concatenate([sc_out, tc_out])


# speedup vs baseline: 1.6157x; 1.2799x over previous
import jax
import jax.numpy as jnp
from jax.experimental import pallas as pl
from jax.experimental.pallas import tpu as pltpu

N_CHANNELS = 4096
N_ENERGIES = 8192
CH = 256
NCH = N_CHANNELS // CH
NBUF = 4


def _copy(tm_hbm, bufs, sems, i, b):
    return pltpu.make_async_copy(
        tm_hbm.at[pl.ds(i * CH, CH), :], bufs.at[b], sems.at[b]
    )


def _stream_kernel(params_ref, energies_ref, tm_hbm, out_ref, bufs, flux_ref, sems):
    for b in range(NBUF):
        _copy(tm_hbm, bufs, sems, b, b).start()

    alpha = params_ref[0, 0]
    norm = params_ref[0, 1]
    oma = 1.0 - alpha
    e_lo = energies_ref[0, :]
    e_hi = energies_ref[1, :]
    flux_ref[...] = (
        (norm / oma) * (jnp.exp(oma * jnp.log(e_hi)) - jnp.exp(oma * jnp.log(e_lo)))
    ).reshape(N_ENERGIES, 1)

    for i in range(NCH):
        b = i % NBUF
        _copy(tm_hbm, bufs, sems, i, b).wait()
        res = jnp.dot(bufs[b], flux_ref[...], preferred_element_type=jnp.float32)
        out_ref[pl.ds(i * CH, CH), :] = jnp.maximum(res, 1e-6)
        if i + NBUF < NCH:
            _copy(tm_hbm, bufs, sems, i + NBUF, b).start()


def kernel(parameters, energies, transfer_matrix):
    params2d = parameters.reshape(1, 2)
    out = pl.pallas_call(
        _stream_kernel,
        in_specs=[
            pl.BlockSpec(memory_space=pltpu.SMEM),
            pl.BlockSpec(memory_space=pltpu.VMEM),
            pl.BlockSpec(memory_space=pltpu.MemorySpace.HBM),
        ],
        out_specs=pl.BlockSpec(memory_space=pltpu.VMEM),
        out_shape=jax.ShapeDtypeStruct((N_CHANNELS, 1), jnp.float32),
        scratch_shapes=[
            pltpu.VMEM((NBUF, CH, N_ENERGIES), jnp.float32),
            pltpu.VMEM((N_ENERGIES, 1), jnp.float32),
            pltpu.SemaphoreType.DMA((NBUF,)),
        ],
    )(params2d, energies, transfer_matrix)
    return out.reshape(N_CHANNELS)
